# baseline (speedup 1.0000x reference)
import jax
import jax.numpy as jnp
from jax import lax
from jax.experimental import pallas as pl
from jax.experimental.pallas import tpu as pltpu
from jax.experimental.pallas import tpu_sc as plsc

_N_USERS = 50000
_N_ITEMS = 50000
_N = _N_USERS + _N_ITEMS
_E = 1600000
_NS = 16

_NP = 100096
_RPT = _NP // _NS
_ZB = 184

_CHUNK = 1024
_G = 128
_GP = _CHUNK // _G
_CR = _CHUNK // 128
_EPT = 100352
_NCHUNK = _EPT // _CHUNK
_EPAD = _EPT * _NS
_ERB = _EPAD // 128
_ER = _E // 128

_PB = 400
_PCHUNKS = _N_USERS // _PB


def _pad_body(c_ref, r_ref, v_ref, co_ref, ro_ref, vo_ref):
    co_ref[0:_ER] = c_ref[...]
    ro_ref[0:_ER] = r_ref[...]
    vo_ref[0:_ER] = v_ref[...]
    co_ref[_ER:_ERB] = jnp.zeros((_ERB - _ER, 128), jnp.int32)
    ro_ref[_ER:_ERB] = jnp.zeros((_ERB - _ER, 128), jnp.int32)
    vo_ref[_ER:_ERB] = jnp.zeros((_ERB - _ER, 128), jnp.float32)


def _sc_body(colr, rowr, valr, ue3, uep3, ie3, iep3,
             u3, i3, e0, e1, e2,
             acc, gsem):
    f32 = jnp.float32
    i32 = jnp.int32
    pl.run_scoped(
        lambda colbuf, rowbuf, valbuf, rows, t0, t1: _sc_inner(
            colr, rowr, valr, ue3, uep3, ie3, iep3,
            u3, i3, e0, e1, e2, acc, gsem,
            colbuf, rowbuf, valbuf, rows, t0, t1),
        pltpu.VMEM((_CR, 128), i32),
        pltpu.VMEM((_CR, 128), i32),
        pltpu.VMEM((_CR, 128), f32),
        pltpu.VMEM((_CHUNK, 16), f32),
        pltpu.VMEM((_ZB, 16), f32),
        pltpu.VMEM((_ZB, 16), f32),
    )


def _sc_inner(colr, rowr, valr, ue3, uep3, ie3, iep3,
              u3, i3, e0, e1, e2, acc, gsem,
              colbuf, rowbuf, valbuf, rows, t0, t1):
    c = lax.axis_index("c")
    s = lax.axis_index("s")
    half = c * _NP
    halfv = lax.broadcast(half, (16,))
    zvec = jnp.zeros((16,), jnp.float32)

    for part, (ea, eb) in enumerate(((ue3, uep3), (ie3, iep3))):

        def pro_chunk(k, ea=ea, eb=eb, part=part):
            r0 = k * _PB
            pltpu.sync_copy(ea.at[pl.ds(r0, _PB), c], rows.at[pl.ds(0, _PB)])
            pltpu.sync_copy(eb.at[pl.ds(r0, _PB), c], rows.at[pl.ds(_PB, _PB)])

            def add_body(i, _):
                rows[i] = rows[i] + rows[_PB + i]
                return 0

            lax.fori_loop(0, _PB, add_body, 0)
            pltpu.sync_copy(rows.at[pl.ds(0, _PB)],
                            e0.at[pl.ds(half + part * _N_USERS + r0, _PB)])

        def pro_loop(j, _):
            pro_chunk(j * _NS + s)
            return 0

        lax.fori_loop(0, _PCHUNKS // _NS, pro_loop, 0)

        @pl.when(s < _PCHUNKS - (_PCHUNKS // _NS) * _NS)
        def _():
            pro_chunk((_PCHUNKS // _NS) * _NS + s)

    def zb_body(i, _):
        t0[i] = zvec
        return 0

    lax.fori_loop(0, _ZB, zb_body, 0)
    for k in range(_RPT // _ZB):
        pltpu.sync_copy(t0, acc.at[pl.ds(s * _RPT + k * _ZB, _ZB)])
    plsc.subcore_barrier()

    ebase = s * (_EPT // 128)
    for layer, src in enumerate((e0, e1, e2)):

        def chunk_body(ch, _, src=src):
            r0 = ebase + ch * _CR
            pltpu.sync_copy(colr.at[pl.ds(r0, _CR)], colbuf)
            pltpu.sync_copy(rowr.at[pl.ds(r0, _CR)], rowbuf)
            pltpu.sync_copy(valr.at[pl.ds(r0, _CR)], valbuf)

            def off_body(i, _):
                jj = i // 8
                tt = i - jj * 8
                colbuf[jj, pl.ds(tt * 16, 16)] = (
                    colbuf[jj, pl.ds(tt * 16, 16)] + halfv)
                return 0

            lax.fori_loop(0, _CHUNK // 16, off_body, 0)

            cps = [pltpu.async_copy(src.at[colbuf.at[j]],
                                    rows.at[pl.ds(j * _G, _G)], gsem)
                   for j in range(_GP)]
            for cp in cps:
                cp.wait()

            def scale_body(g, _):
                jj = g // 8
                tt = g - jj * 8
                vv = valbuf[jj, pl.ds(tt * 16, 16)]
                base = g * 16
                for e in range(16):
                    sv = lax.broadcast(vv[e], (16,))
                    rows[base + e] = rows[base + e] * sv
                return 0

            lax.fori_loop(0, _CHUNK // 16, scale_body, 0)
            for j in range(_GP):
                pltpu.sync_copy(rows.at[pl.ds(j * _G, _G)],
                                acc.at[rowbuf.at[j]], add=True)
            return 0

        lax.fori_loop(0, _NCHUNK, chunk_body, 0)
        plsc.subcore_barrier()

        if layer == 0:
            def wb1_body(k, _):
                b = s * _RPT + k * _ZB
                pltpu.sync_copy(acc.at[pl.ds(b, _ZB)], t0)
                pltpu.sync_copy(t0, e1.at[pl.ds(half + b, _ZB)])
                return 0

            lax.fori_loop(0, _RPT // _ZB, wb1_body, 0)
            plsc.subcore_barrier()
        elif layer == 1:
            def wb2_body(k, _):
                b = s * _RPT + k * _ZB
                pltpu.sync_copy(acc.at[pl.ds(b, _ZB)], t0)
                pltpu.sync_copy(e1.at[pl.ds(half + b, _ZB)], t1)

                def sub_body(i, _):
                    t0[i] = t0[i] - t1[i]
                    return 0

                lax.fori_loop(0, _ZB, sub_body, 0)
                pltpu.sync_copy(t0, e2.at[pl.ds(half + b, _ZB)])
                return 0

            lax.fori_loop(0, _RPT // _ZB, wb2_body, 0)
            plsc.subcore_barrier()

    for part, out in enumerate((u3, i3)):

        def epi_chunk(k, out=out, part=part):
            r0 = k * _PB
            b = part * _N_USERS + r0
            pltpu.sync_copy(e0.at[pl.ds(half + b, _PB)], rows.at[pl.ds(0, _PB)])
            pltpu.sync_copy(acc.at[pl.ds(b, _PB)], rows.at[pl.ds(_PB, _PB)])

            def mean_body(i, _):
                rows[i] = (rows[i] + rows[_PB + i]) * 0.25
                return 0

            lax.fori_loop(0, _PB, mean_body, 0)
            pltpu.sync_copy(rows.at[pl.ds(0, _PB)],
                            out.at[pl.ds(r0, _PB), c])

        def epi_loop(j, _):
            epi_chunk(j * _NS + s)
            return 0

        lax.fori_loop(0, _PCHUNKS // _NS, epi_loop, 0)

        @pl.when(s < _PCHUNKS - (_PCHUNKS // _NS) * _NS)
        def _():
            epi_chunk((_PCHUNKS // _NS) * _NS + s)


def kernel(adj_index, adj_values, user_emb, user_emb_pre, item_emb, item_emb_pre):
    f32 = jnp.float32
    i32 = jnp.int32

    colr, rowr, valr = pl.pallas_call(
        _pad_body,
        out_shape=[
            jax.ShapeDtypeStruct((_ERB, 128), i32),
            jax.ShapeDtypeStruct((_ERB, 128), i32),
            jax.ShapeDtypeStruct((_ERB, 128), f32),
        ],
    )(adj_index[1].reshape(_ER, 128),
      adj_index[0].reshape(_ER, 128),
      adj_values.reshape(_ER, 128))

    ue3 = user_emb.reshape(_N_USERS, 2, 16)
    uep3 = user_emb_pre.reshape(_N_USERS, 2, 16)
    ie3 = item_emb.reshape(_N_ITEMS, 2, 16)
    iep3 = item_emb_pre.reshape(_N_ITEMS, 2, 16)

    mesh = plsc.VectorSubcoreMesh(core_axis_name="c", subcore_axis_name="s")
    outs = pl.kernel(
        _sc_body,
        out_type=[
            jax.ShapeDtypeStruct((_N_USERS, 2, 16), f32),
            jax.ShapeDtypeStruct((_N_ITEMS, 2, 16), f32),
            jax.ShapeDtypeStruct((2 * _NP, 16), f32),
            jax.ShapeDtypeStruct((2 * _NP, 16), f32),
            jax.ShapeDtypeStruct((2 * _NP, 16), f32),
        ],
        mesh=mesh,
        compiler_params=pltpu.CompilerParams(use_tc_tiling_on_sc=False),
        scratch_types=[
            pltpu.VMEM_SHARED((_NP, 16), f32),
            pltpu.SemaphoreType.DMA,
        ],
    )(colr, rowr, valr, ue3, uep3, ie3, iep3)
    u3, i3 = outs[0], outs[1]
    return (u3.reshape(_N_USERS, 32), i3.reshape(_N_ITEMS, 32))


# speedup vs baseline: 1.1448x; 1.1448x over previous
import jax
import jax.numpy as jnp
from jax import lax
from jax.experimental import pallas as pl
from jax.experimental.pallas import tpu as pltpu
from jax.experimental.pallas import tpu_sc as plsc

_N_USERS = 50000
_N_ITEMS = 50000
_N = _N_USERS + _N_ITEMS
_E = 1600000
_NS = 16

_NP = 100096
_RPT = _NP // _NS
_ZB = 184

_CHUNK = 1024
_G = 128
_GP = _CHUNK // _G
_CR = _CHUNK // 128
_EPT = 100352
_NCHUNK = _EPT // _CHUNK
_EPAD = _EPT * _NS
_ERB = _EPAD // 128
_ER = _E // 128

_PB = 200
_PCHUNKS = _N_USERS // _PB


def _pack_body(c_ref, r_ref, v_ref, o_ref):
    zi = jnp.zeros((_ERB - _ER, 128), jnp.int32)
    o_ref[0:_ER] = c_ref[...]
    o_ref[_ER:_ERB] = zi
    o_ref[_ERB:_ERB + _ER] = r_ref[...]
    o_ref[_ERB + _ER:2 * _ERB] = zi
    o_ref[2 * _ERB:2 * _ERB + _ER] = v_ref[...]
    o_ref[2 * _ERB + _ER:3 * _ERB] = zi


def _add_body(a_ref, b_ref, o_ref):
    o_ref[...] = a_ref[...] + b_ref[...]


def _sc_body(edges, e0, uo3, e12, acc, gsem):
    f32 = jnp.float32
    i32 = jnp.int32
    pl.run_scoped(
        lambda colbuf, rowbuf, valbuf, rows, t0, t1: _sc_inner(
            edges, e0, uo3, e12, acc, gsem,
            colbuf, rowbuf, valbuf, rows, t0, t1),
        pltpu.VMEM((_CR, 128), i32),
        pltpu.VMEM((_CR, 128), i32),
        pltpu.VMEM((_CR, 128), i32),
        pltpu.VMEM((_CHUNK, 16), f32),
        pltpu.VMEM((_ZB, 16), f32),
        pltpu.VMEM((_ZB, 16), f32),
    )


def _sc_inner(edges, e0, uo3, e12, acc, gsem,
              colbuf, rowbuf, valbuf, rows, t0, t1):
    c = lax.axis_index("c")
    s = lax.axis_index("s")
    half = c * _NP
    zvec = jnp.zeros((16,), jnp.float32)

    def zb_body(i, _):
        t0[i] = zvec
        return 0

    lax.fori_loop(0, _ZB, zb_body, 0)
    for k in range(_RPT // _ZB):
        pltpu.sync_copy(t0, acc.at[pl.ds(s * _RPT + k * _ZB, _ZB)])
    plsc.subcore_barrier()

    ebase = s * (_EPT // 128)
    srcs = (e0, e12, e12)
    for layer in range(3):
        src = srcs[layer]
        mul2 = layer == 0
        offv = lax.broadcast(c if mul2 else (layer - 1) * 2 * _NP + half, (16,))

        def chunk_body(ch, _, src=src, mul2=mul2, offv=offv):
            r0 = ebase + ch * _CR
            pltpu.sync_copy(edges.at[pl.ds(r0, _CR)], colbuf)
            pltpu.sync_copy(edges.at[pl.ds(_ERB + r0, _CR)], rowbuf)
            pltpu.sync_copy(edges.at[pl.ds(2 * _ERB + r0, _CR)], valbuf)

            def off_body(i, _):
                jj = i // 8
                tt = i - jj * 8
                x = colbuf[jj, pl.ds(tt * 16, 16)]
                if mul2:
                    x = x + x
                colbuf[jj, pl.ds(tt * 16, 16)] = x + offv
                return 0

            lax.fori_loop(0, _CHUNK // 16, off_body, 0)

            cps = [pltpu.async_copy(src.at[colbuf.at[j]],
                                    rows.at[pl.ds(j * _G, _G)], gsem)
                   for j in range(_GP)]
            for cp in cps:
                cp.wait()

            def scale_body(g, _):
                jj = g // 8
                tt = g - jj * 8
                vv = plsc.bitcast(valbuf[jj, pl.ds(tt * 16, 16)], jnp.float32)
                base = g * 16
                for e in range(16):
                    sv = lax.broadcast(vv[e], (16,))
                    rows[base + e] = rows[base + e] * sv
                return 0

            lax.fori_loop(0, _CHUNK // 16, scale_body, 0)
            for j in range(_GP):
                pltpu.sync_copy(rows.at[pl.ds(j * _G, _G)],
                                acc.at[rowbuf.at[j]], add=True)
            return 0

        lax.fori_loop(0, _NCHUNK, chunk_body, 0)
        plsc.subcore_barrier()

        if layer == 0:
            def wb1_body(k, _):
                b = s * _RPT + k * _ZB
                pltpu.sync_copy(acc.at[pl.ds(b, _ZB)], t0)
                pltpu.sync_copy(t0, e12.at[pl.ds(half + b, _ZB)])
                return 0

            lax.fori_loop(0, _RPT // _ZB, wb1_body, 0)
            plsc.subcore_barrier()
        elif layer == 1:
            def wb2_body(k, _):
                b = s * _RPT + k * _ZB
                pltpu.sync_copy(acc.at[pl.ds(b, _ZB)], t0)
                pltpu.sync_copy(e12.at[pl.ds(half + b, _ZB)], t1)

                def sub_body(i, _):
                    t0[i] = t0[i] - t1[i]
                    return 0

                lax.fori_loop(0, _ZB, sub_body, 0)
                pltpu.sync_copy(t0, e12.at[pl.ds(2 * _NP + half + b, _ZB)])
                return 0

            lax.fori_loop(0, _RPT // _ZB, wb2_body, 0)
            plsc.subcore_barrier()

    def epi_chunk(k):
        r0 = k * _PB
        pltpu.sync_copy(e0.at[pl.ds(2 * r0, 2 * _PB)], rows.at[pl.ds(0, 2 * _PB)])
        pltpu.sync_copy(acc.at[pl.ds(r0, _PB)], rows.at[pl.ds(2 * _PB, _PB)])

        def mean_body(i, _):
            rows[3 * _PB + i] = (rows[2 * i + c] + rows[2 * _PB + i]) * 0.25
            return 0

        lax.fori_loop(0, _PB, mean_body, 0)
        pltpu.sync_copy(rows.at[pl.ds(3 * _PB, _PB)],
                        uo3.at[pl.ds(r0, _PB), c])

    nfull = (2 * _PCHUNKS) // _NS

    def epi_loop(j, _):
        epi_chunk(j * _NS + s)
        return 0

    lax.fori_loop(0, nfull, epi_loop, 0)

    rem = 2 * _PCHUNKS - nfull * _NS
    if rem:
        @pl.when(s < rem)
        def _():
            epi_chunk(nfull * _NS + s)


def kernel(adj_index, adj_values, user_emb, user_emb_pre, item_emb, item_emb_pre):
    f32 = jnp.float32
    i32 = jnp.int32

    vbits = lax.bitcast_convert_type(adj_values, i32)
    edges = pl.pallas_call(
        _pack_body,
        out_shape=jax.ShapeDtypeStruct((3 * _ERB, 128), i32),
    )(adj_index[1].reshape(_ER, 128),
      adj_index[0].reshape(_ER, 128),
      vbits.reshape(_ER, 128))

    nr = _N * 32 // 128
    allemb = jnp.concatenate([user_emb, item_emb], axis=0).reshape(nr, 128)
    allpre = jnp.concatenate([user_emb_pre, item_emb_pre], axis=0).reshape(nr, 128)
    blk = 1000
    ego = pl.pallas_call(
        _add_body,
        grid=(nr // blk,),
        in_specs=[pl.BlockSpec((blk, 128), lambda r: (r, 0)),
                  pl.BlockSpec((blk, 128), lambda r: (r, 0))],
        out_specs=pl.BlockSpec((blk, 128), lambda r: (r, 0)),
        out_shape=jax.ShapeDtypeStruct((nr, 128), f32),
    )(allemb, allpre)
    e0 = ego.reshape(2 * _N, 16)

    mesh = plsc.VectorSubcoreMesh(core_axis_name="c", subcore_axis_name="s")
    uo3, _unused_e12 = pl.kernel(
        _sc_body,
        out_type=[
            jax.ShapeDtypeStruct((_N, 2, 16), f32),
            jax.ShapeDtypeStruct((4 * _NP, 16), f32),
        ],
        mesh=mesh,
        compiler_params=pltpu.CompilerParams(use_tc_tiling_on_sc=False,
                                             needs_layout_passes=False),
        scratch_types=[
            pltpu.VMEM_SHARED((_NP, 16), f32),
            pltpu.SemaphoreType.DMA,
        ],
    )(edges, e0)
    mean = uo3.reshape(_N, 32)
    return (mean[:_N_USERS], mean[_N_USERS:])


# speedup vs baseline: 1.8007x; 1.5729x over previous
import jax
import jax.numpy as jnp
from jax import lax
from jax.experimental import pallas as pl
from jax.experimental.pallas import tpu as pltpu
from jax.experimental.pallas import tpu_sc as plsc

_N_USERS = 50000
_N_ITEMS = 50000
_N = _N_USERS + _N_ITEMS
_E = 1600000
_NS = 16

_NP = 100096
_RPT = _NP // _NS
_ZB = 184

_CHUNK = 1024
_G = 128
_GP = _CHUNK // _G
_CR = _CHUNK // 128
_EPT = 100352
_NCHUNK = _EPT // _CHUNK
_EPAD = _EPT * _NS
_ERB = _EPAD // 128
_ER = _E // 128

_PB = 200
_PCHUNKS = _N_USERS // _PB


def _pack_body(c_ref, r_ref, v_ref, o_ref):
    zi = jnp.zeros((_ERB - _ER, 128), jnp.int32)
    o_ref[0:_ER] = c_ref[...]
    o_ref[_ER:_ERB] = zi
    o_ref[_ERB:_ERB + _ER] = r_ref[...]
    o_ref[_ERB + _ER:2 * _ERB] = zi
    o_ref[2 * _ERB:2 * _ERB + _ER] = v_ref[...]
    o_ref[2 * _ERB + _ER:3 * _ERB] = zi


def _add_body(a_ref, b_ref, o_ref):
    o_ref[...] = a_ref[...] + b_ref[...]


def _sc_body(edges, e0, uo3, e12, acc, gsem, dsem, ssem):
    f32 = jnp.float32
    i32 = jnp.int32
    pl.run_scoped(
        lambda colA, rowA, valA, colB, rowB, valB, rows, t0, t1: _sc_inner(
            edges, e0, uo3, e12, acc, gsem, dsem, ssem,
            colA, rowA, valA, colB, rowB, valB, rows, t0, t1),
        pltpu.VMEM((_CR, 128), i32),
        pltpu.VMEM((_CR, 128), i32),
        pltpu.VMEM((_CR, 128), i32),
        pltpu.VMEM((_CR, 128), i32),
        pltpu.VMEM((_CR, 128), i32),
        pltpu.VMEM((_CR, 128), i32),
        pltpu.VMEM((_CHUNK, 16), f32),
        pltpu.VMEM((_ZB, 16), f32),
        pltpu.VMEM((_ZB, 16), f32),
    )


def _sc_inner(edges, e0, uo3, e12, acc, gsem, dsem, ssem,
              colA, rowA, valA, colB, rowB, valB, rows, t0, t1):
    c = lax.axis_index("c")
    s = lax.axis_index("s")
    half = c * _NP
    zvec = jnp.zeros((16,), jnp.float32)

    def zb_body(i, _):
        t0[i] = zvec
        return 0

    lax.fori_loop(0, _ZB, zb_body, 0)
    for k in range(_RPT // _ZB):
        pltpu.sync_copy(t0, acc.at[pl.ds(s * _RPT + k * _ZB, _ZB)])
    plsc.subcore_barrier()

    ebase = s * (_EPT // 128)
    srcs = (e0, e12, e12)
    isets = ((colA, rowA, valA), (colB, rowB, valB))

    def idx_issue(ch, iset):
        cb, rb, vb = iset
        r0 = ebase + ch * _CR
        pltpu.async_copy(edges.at[pl.ds(r0, _CR)], cb, dsem)
        pltpu.async_copy(edges.at[pl.ds(_ERB + r0, _CR)], rb, dsem)
        pltpu.async_copy(edges.at[pl.ds(2 * _ERB + r0, _CR)], vb, dsem)

    def idx_wait(ch, iset):
        cb, rb, vb = iset
        r0 = ebase + ch * _CR
        pltpu.make_async_copy(edges.at[pl.ds(r0, _CR)], cb, dsem).wait()
        pltpu.make_async_copy(edges.at[pl.ds(_ERB + r0, _CR)], rb, dsem).wait()
        pltpu.make_async_copy(edges.at[pl.ds(2 * _ERB + r0, _CR)], vb, dsem).wait()

    def drain_scatters():
        for _j in range(_GP):
            pltpu.make_async_copy(e12.at[pl.ds(0, _G)],
                                  acc.at[pl.ds(0, _G)], ssem).wait()

    for layer in range(3):
        src = srcs[layer]
        mul2 = layer == 0
        offv = lax.broadcast(c if mul2 else (layer - 1) * 2 * _NP + half, (16,))

        def process(ch, iset, first, last, src=src, mul2=mul2, offv=offv):
            cb, rb, vb = iset
            idx_wait(ch, iset)

            def off_body(i, _):
                jj = i // 8
                tt = i - jj * 8
                x = cb[jj, pl.ds(tt * 16, 16)]
                if mul2:
                    x = x + x
                cb[jj, pl.ds(tt * 16, 16)] = x + offv
                return 0

            lax.fori_loop(0, _CHUNK // 16, off_body, 0)

            if first:
                drained = True
            else:
                @pl.when(ch > 0)
                def _():
                    drain_scatters()

            cps = [pltpu.async_copy(src.at[cb.at[j]],
                                    rows.at[pl.ds(j * _G, _G)], gsem)
                   for j in range(_GP)]

            if not last:
                @pl.when(ch < _NCHUNK - 1)
                def _():
                    idx_issue(ch + 1, isets[1] if iset is isets[0] else isets[0])

            for j in range(_GP):
                cps[j].wait()
                g0 = j * 8

                def scale_body(g, _):
                    jj = g // 8
                    tt = g - jj * 8
                    vv = plsc.bitcast(vb[jj, pl.ds(tt * 16, 16)], jnp.float32)
                    base = g * 16
                    for e in range(16):
                        sv = lax.broadcast(vv[e], (16,))
                        rows[base + e] = rows[base + e] * sv
                    return 0

                lax.fori_loop(g0, g0 + 8, scale_body, 0)
                pltpu.async_copy(rows.at[pl.ds(j * _G, _G)],
                                 acc.at[rb.at[j]], ssem, add=True)

        idx_issue(0, isets[0])

        def pair_body(m, _):
            process(2 * m, isets[0], False, False)
            process(2 * m + 1, isets[1], False, False)
            return 0

        lax.fori_loop(0, _NCHUNK // 2, pair_body, 0)
        drain_scatters()
        plsc.subcore_barrier()

        if layer == 0:
            def wb1_body(k, _):
                b = s * _RPT + k * _ZB
                pltpu.sync_copy(acc.at[pl.ds(b, _ZB)], t0)
                pltpu.sync_copy(t0, e12.at[pl.ds(half + b, _ZB)])
                return 0

            lax.fori_loop(0, _RPT // _ZB, wb1_body, 0)
            plsc.subcore_barrier()
        elif layer == 1:
            def wb2_body(k, _):
                b = s * _RPT + k * _ZB
                pltpu.sync_copy(acc.at[pl.ds(b, _ZB)], t0)
                pltpu.sync_copy(e12.at[pl.ds(half + b, _ZB)], t1)

                def sub_body(i, _):
                    t0[i] = t0[i] - t1[i]
                    return 0

                lax.fori_loop(0, _ZB, sub_body, 0)
                pltpu.sync_copy(t0, e12.at[pl.ds(2 * _NP + half + b, _ZB)])
                return 0

            lax.fori_loop(0, _RPT // _ZB, wb2_body, 0)
            plsc.subcore_barrier()

    def epi_chunk(k):
        r0 = k * _PB
        pltpu.sync_copy(e0.at[pl.ds(2 * r0, 2 * _PB)], rows.at[pl.ds(0, 2 * _PB)])
        pltpu.sync_copy(acc.at[pl.ds(r0, _PB)], rows.at[pl.ds(2 * _PB, _PB)])

        def mean_body(i, _):
            rows[3 * _PB + i] = (rows[2 * i + c] + rows[2 * _PB + i]) * 0.25
            return 0

        lax.fori_loop(0, _PB, mean_body, 0)
        pltpu.sync_copy(rows.at[pl.ds(3 * _PB, _PB)],
                        uo3.at[pl.ds(r0, _PB), c])

    nfull = (2 * _PCHUNKS) // _NS

    def epi_loop(j, _):
        epi_chunk(j * _NS + s)
        return 0

    lax.fori_loop(0, nfull, epi_loop, 0)

    rem = 2 * _PCHUNKS - nfull * _NS
    if rem:
        @pl.when(s < rem)
        def _():
            epi_chunk(nfull * _NS + s)


def kernel(adj_index, adj_values, user_emb, user_emb_pre, item_emb, item_emb_pre):
    f32 = jnp.float32
    i32 = jnp.int32

    vbits = lax.bitcast_convert_type(adj_values, i32)
    edges = pl.pallas_call(
        _pack_body,
        out_shape=jax.ShapeDtypeStruct((3 * _ERB, 128), i32),
    )(adj_index[1].reshape(_ER, 128),
      adj_index[0].reshape(_ER, 128),
      vbits.reshape(_ER, 128))

    nr = _N * 32 // 128
    allemb = jnp.concatenate([user_emb, item_emb], axis=0).reshape(nr, 128)
    allpre = jnp.concatenate([user_emb_pre, item_emb_pre], axis=0).reshape(nr, 128)
    blk = 1000
    ego = pl.pallas_call(
        _add_body,
        grid=(nr // blk,),
        in_specs=[pl.BlockSpec((blk, 128), lambda r: (r, 0)),
                  pl.BlockSpec((blk, 128), lambda r: (r, 0))],
        out_specs=pl.BlockSpec((blk, 128), lambda r: (r, 0)),
        out_shape=jax.ShapeDtypeStruct((nr, 128), f32),
    )(allemb, allpre)
    e0 = ego.reshape(2 * _N, 16)

    mesh = plsc.VectorSubcoreMesh(core_axis_name="c", subcore_axis_name="s")
    uo3, _unused_e12 = pl.kernel(
        _sc_body,
        out_type=[
            jax.ShapeDtypeStruct((_N, 2, 16), f32),
            jax.ShapeDtypeStruct((4 * _NP, 16), f32),
        ],
        mesh=mesh,
        compiler_params=pltpu.CompilerParams(use_tc_tiling_on_sc=False,
                                             needs_layout_passes=False),
        scratch_types=[
            pltpu.VMEM_SHARED((_NP, 16), f32),
            pltpu.SemaphoreType.DMA,
            pltpu.SemaphoreType.DMA,
            pltpu.SemaphoreType.DMA,
        ],
    )(edges, e0)
    mean = uo3.reshape(_N, 32)
    return (mean[:_N_USERS], mean[_N_USERS:])
